# Initial kernel scaffold; baseline (speedup 1.0000x reference)
import jax
import jax.numpy as jnp
from jax.experimental import pallas as pl


def kernel(x, edge_index, W1l, b1l, W1r, W2l, b2l, W2r, Wout, bout):
    raise NotImplementedError("write your pallas kernel here")


# speedup vs baseline: 19.1403x; 19.1403x over previous
import functools

import jax
import jax.numpy as jnp
from jax import lax
from jax.experimental import pallas as pl
from jax.experimental.pallas import tpu as pltpu
from jax.experimental.pallas import tpu_sc as plsc

N = 10000
NP = 10240
DF = 128
E = 320000
NW = 32
CH = 80
EP = NW * CH * 128
RT = NP // 16
BI = 20


@functools.cache
def _build_sc_feat_agg():
    mesh = plsc.VectorSubcoreMesh(core_axis_name="c", subcore_axis_name="s")

    @functools.partial(
        pl.kernel,
        out_type=[
            jax.ShapeDtypeStruct((2, NP, DF), jnp.float32),
            jax.ShapeDtypeStruct((2, NP // 128, 128), jnp.float32),
        ],
        mesh=mesh,
        compiler_params=pltpu.CompilerParams(use_tc_tiling_on_sc=False,
                                             needs_layout_passes=False),
        scratch_types=[
            pltpu.VMEM((BI, 128), jnp.int32),
            pltpu.VMEM((BI, 128), jnp.int32),
            pltpu.VMEM((128, DF), jnp.float32),
            pltpu.VMEM((128, DF), jnp.float32),
            pltpu.VMEM((NP // 128, 128), jnp.float32),
            pltpu.VMEM((NP // 128,), jnp.int32),
            pltpu.VMEM_SHARED((NP, DF), jnp.float32),
            pltpu.VMEM_SHARED((NP // 128, 128), jnp.float32),
            pltpu.SemaphoreType.DMA,
            pltpu.SemaphoreType.DMA,
            pltpu.SemaphoreType.DMA,
            pltpu.SemaphoreType.DMA,
        ],
    )
    def sc_feat_agg(xp_hbm, src_hbm, dst_hbm, agg_out, cnt_out, srcv, dstv,
                    gbuf0, gbuf1, cntv, riota, agg_sh, cnt_sh, sem0, sem1,
                    sem2, sem3):
        cid = lax.axis_index("c")
        sid = lax.axis_index("s")
        wid = sid * 2 + cid

        z16 = jnp.zeros((16,), jnp.float32)
        ones16 = jnp.ones((16,), jnp.float32)

        def _zrow(r, carry):
            for k in range(DF // 16):
                gbuf0[r, pl.ds(k * 16, 16)] = z16
            return carry

        lax.fori_loop(0, 128, _zrow, 0)

        def _zcnt(r, carry):
            for k in range(8):
                cntv[r, pl.ds(k * 16, 16)] = z16
            return carry

        lax.fori_loop(0, NP // 128, _zcnt, 0)
        for k in range(NP // 128 // 16):
            riota[pl.ds(k * 16, 16)] = lax.iota(jnp.int32, 16) + k * 16
        for kk in range(RT // 128):
            pltpu.sync_copy(gbuf0, agg_sh.at[pl.ds(sid * RT + kk * 128, 128)])
        pltpu.sync_copy(cntv.at[pl.ds(sid * 5, 5)],
                        cnt_sh.at[pl.ds(sid * 5, 5)])

        plsc.subcore_barrier()

        def _gstart(j, buf, sa, sb):
            del sb
            pltpu.async_copy(xp_hbm.at[srcv.at[j]], buf, sa)

        def _gwait(j, buf, sa, sb):
            del sb
            pltpu.make_async_copy(xp_hbm.at[srcv.at[j]], buf, sa).wait()

        def _block(b, carry):
            pltpu.sync_copy(src_hbm.at[wid, pl.ds(b * BI, BI)], srcv)
            pltpu.sync_copy(dst_hbm.at[wid, pl.ds(b * BI, BI)], dstv)
            _gstart(0, gbuf0, sem0, sem2)

            def _chunk2(i, c2):
                j0 = 2 * i
                _gstart(j0 + 1, gbuf1, sem1, sem3)
                _gwait(j0, gbuf0, sem0, sem2)
                pltpu.sync_copy(gbuf0, agg_sh.at[dstv.at[j0]], add=True)

                @pl.when(j0 + 2 < BI)
                def _():
                    _gstart(j0 + 2, gbuf0, sem0, sem2)

                for k in range(8):
                    d16 = dstv[j0, pl.ds(k * 16, 16)]
                    plsc.addupdate_scatter(
                        cntv, [d16 >> 7, d16 & 127], ones16)

                _gwait(j0 + 1, gbuf1, sem1, sem3)
                pltpu.sync_copy(gbuf1, agg_sh.at[dstv.at[j0 + 1]], add=True)

                for k in range(8):
                    d16 = dstv[j0 + 1, pl.ds(k * 16, 16)]
                    plsc.addupdate_scatter(
                        cntv, [d16 >> 7, d16 & 127], ones16)
                return c2

            lax.fori_loop(0, BI // 2, _chunk2, 0)
            return carry

        lax.fori_loop(0, CH // BI, _block, 0)

        pltpu.sync_copy(cntv, cnt_sh.at[riota], add=True)
        plsc.subcore_barrier()
        pltpu.sync_copy(agg_sh.at[pl.ds(sid * RT, RT)],
                        agg_out.at[cid, pl.ds(sid * RT, RT)])
        pltpu.sync_copy(cnt_sh.at[pl.ds(sid * 5, 5)],
                        cnt_out.at[cid, pl.ds(sid * 5, 5)])

    return sc_feat_agg


@functools.cache
def _build_sc_scalar_agg():
    mesh = plsc.VectorSubcoreMesh(core_axis_name="c", subcore_axis_name="s")

    @functools.partial(
        pl.kernel,
        out_type=jax.ShapeDtypeStruct((2, NP // 128, 128), jnp.float32),
        mesh=mesh,
        compiler_params=pltpu.CompilerParams(use_tc_tiling_on_sc=False,
                                             needs_layout_passes=False),
        scratch_types=[
            pltpu.VMEM((CH, 128), jnp.int32),
            pltpu.VMEM((CH, 128), jnp.int32),
            pltpu.VMEM((NP,), jnp.float32),
            pltpu.VMEM((NP // 128, 128), jnp.float32),
            pltpu.VMEM((NP // 128,), jnp.int32),
            pltpu.VMEM_SHARED((NP // 128, 128), jnp.float32),
        ],
    )
    def sc_scalar_agg(s_hbm, src_hbm, dst_hbm, m_out, srcv, dstv, sv, msum,
                      riota, m_sh):
        cid = lax.axis_index("c")
        sid = lax.axis_index("s")
        wid = sid * 2 + cid

        pltpu.sync_copy(src_hbm.at[wid], srcv)
        pltpu.sync_copy(dst_hbm.at[wid], dstv)
        pltpu.sync_copy(s_hbm, sv)

        z16 = jnp.zeros((16,), jnp.float32)

        def _zrow(r, carry):
            for k in range(8):
                msum[r, pl.ds(k * 16, 16)] = z16
            return carry

        lax.fori_loop(0, NP // 128, _zrow, 0)
        for k in range(NP // 128 // 16):
            riota[pl.ds(k * 16, 16)] = lax.iota(jnp.int32, 16) + k * 16
        pltpu.sync_copy(msum.at[pl.ds(sid * 5, 5)],
                        m_sh.at[pl.ds(sid * 5, 5)])

        plsc.subcore_barrier()

        def _chunk(j, carry):
            for k in range(8):
                sidx = srcv[j, pl.ds(k * 16, 16)]
                d16 = dstv[j, pl.ds(k * 16, 16)]
                val = plsc.load_gather(sv, [sidx])
                plsc.addupdate_scatter(msum, [d16 >> 7, d16 & 127], val)
            return carry

        lax.fori_loop(0, CH, _chunk, 0)

        pltpu.sync_copy(msum, m_sh.at[riota], add=True)
        plsc.subcore_barrier()
        pltpu.sync_copy(m_sh.at[pl.ds(sid * 5, 5)],
                        m_out.at[cid, pl.ds(sid * 5, 5)])

    return sc_scalar_agg


def _tc_layer1_body(agg_ref, cnt_ref, x_ref, W1l_ref, b1l_ref, W1r_ref,
                    W2l_ref, W2r_ref, Wout_ref, s_ref, t_ref):
    feat = agg_ref[0] + agg_ref[1]
    cnt = cnt_ref[0] + cnt_ref[1]
    mean = feat * (1.0 / jnp.maximum(cnt, 1.0))
    x = x_ref[...]
    cdims = (((1,), (1,)), ((), ()))
    z = (lax.dot_general(mean, W1l_ref[...], cdims,
                         preferred_element_type=jnp.float32,
                         precision=lax.Precision.HIGHEST)
         + b1l_ref[...]
         + lax.dot_general(x, W1r_ref[...], cdims,
                           preferred_element_type=jnp.float32,
                         precision=lax.Precision.HIGHEST))
    h = jnp.maximum(z, 0.0)
    v = jnp.dot(Wout_ref[...], W2l_ref[...],
                preferred_element_type=jnp.float32,
                         precision=lax.Precision.HIGHEST)
    u = jnp.dot(Wout_ref[...], W2r_ref[...],
                preferred_element_type=jnp.float32,
                         precision=lax.Precision.HIGHEST)
    s_ref[...] = lax.dot_general(h, v, cdims,
                                 preferred_element_type=jnp.float32,
                         precision=lax.Precision.HIGHEST)
    t_ref[...] = lax.dot_general(h, u, cdims,
                                 preferred_element_type=jnp.float32,
                         precision=lax.Precision.HIGHEST)


_BLK = 1024


def _tc_layer1(agg, cnt3, x2, W1l, b1l2, W1r, W2l, W2r, Wout):
    grid = (NP // _BLK,)
    wspec = pl.BlockSpec((128, 128), lambda i: (0, 0))
    rspec = pl.BlockSpec((1, 128), lambda i: (0, 0))
    return pl.pallas_call(
        _tc_layer1_body,
        grid=grid,
        in_specs=[
            pl.BlockSpec((2, _BLK, DF), lambda i: (0, i, 0)),
            pl.BlockSpec((2, _BLK, 1), lambda i: (0, i, 0)),
            pl.BlockSpec((_BLK, 128), lambda i: (i, 0)),
            wspec, rspec, wspec, wspec, wspec, rspec,
        ],
        out_specs=[
            pl.BlockSpec((_BLK, 1), lambda i: (i, 0)),
            pl.BlockSpec((_BLK, 1), lambda i: (i, 0)),
        ],
        out_shape=[
            jax.ShapeDtypeStruct((NP, 1), jnp.float32),
            jax.ShapeDtypeStruct((NP, 1), jnp.float32),
        ],
    )(agg, cnt3, x2, W1l, b1l2, W1r, W2l, W2r, Wout)


def _tc_final_body(m_ref, cnt_ref, t_ref, b2l_ref, Wout_ref, bout_ref,
                   out_ref):
    c = jnp.sum(Wout_ref[...] * b2l_ref[...]) + jnp.sum(bout_ref[...])
    msum = m_ref[0] + m_ref[1]
    cnt = cnt_ref[0] + cnt_ref[1]
    out_ref[...] = msum * (1.0 / jnp.maximum(cnt, 1.0)) + t_ref[...] + c


def _tc_final(m2, cnt2, t2, b2l2, Wout, bout2):
    return pl.pallas_call(
        _tc_final_body,
        out_shape=jax.ShapeDtypeStruct((NP // 128, 128), jnp.float32),
    )(m2, cnt2, t2, b2l2, Wout, bout2)


def kernel(x, edge_index, W1l, b1l, W1r, W2l, b2l, W2r, Wout, bout):
    src = edge_index[0].astype(jnp.int32)
    dst = edge_index[1].astype(jnp.int32)
    npad = EP - E
    pad_src = (jnp.arange(npad, dtype=jnp.int32) * 997) % N
    pad_dst = N + (jnp.arange(npad, dtype=jnp.int32) % (NP - N))
    src3 = jnp.concatenate([src, pad_src]).reshape(NW, CH, 128)
    dst3 = jnp.concatenate([dst, pad_dst]).reshape(NW, CH, 128)

    xp = jnp.zeros((NP, DF), jnp.float32)
    xp = xp.at[:N].set(x)

    agg, cnt = _build_sc_feat_agg()(xp, src3, dst3)

    s, t = _tc_layer1(agg, cnt.reshape(2, NP, 1), xp, W1l,
                      b1l.reshape(1, 128), W1r, W2l, W2r,
                      Wout)

    m = _build_sc_scalar_agg()(s.reshape(NP), src3, dst3)

    cnt2 = cnt.reshape(2, NP // 128, 128)
    out = _tc_final(m, cnt2,
                    t.reshape(NP // 128, 128), b2l.reshape(1, 128), Wout,
                    bout.reshape(1, 1))
    return out.reshape(NP)[:N]
